# baseline (speedup 1.0000x reference)
import functools

import jax
import jax.numpy as jnp
from jax import lax
from jax.experimental import pallas as pl
from jax.experimental.pallas import tpu as pltpu
from jax.experimental.pallas import tpu_sc as plsc

N = 10000
EMB = 128
HID = 128
NREL = 3
NL = 2
BATCH = 100

NSC = 2
NSUB = 16
NW = NSC * NSUB
L = 16

CH = 128
PT = 5120
EP = NW * PT
NCH = PT // CH
TOTCH = EP // CH
NB = 2

RELSZ = NREL * N
PAD_RD = RELSZ
DEG_OFF = 30720
HSZ = 40960
NPAD = 10240
ROWS_T = NPAD // NSUB

F0 = 52
F1 = (2 * NCH) - F0

_mesh = plsc.VectorSubcoreMesh(core_axis_name="c", subcore_axis_name="s")


def _tile_chunks(cid, sid):
    cbase = jnp.where(cid == 0, sid * F0, NSUB * F0 + sid * F1)
    nch_t = jnp.where(cid == 0, F0, F1)
    return cbase, nch_t


def _wid(cid, sid):
    return sid * NSC + cid


def _dyn_gather(a, idx):
    dn = lax.GatherDimensionNumbers(
        offset_dims=(), collapsed_slice_dims=(0,), start_index_map=(0,)
    )
    return lax.gather(
        a, idx[:, None], dn, slice_sizes=(1,),
        mode=lax.GatherScatterMode.PROMISE_IN_BOUNDS,
    )


@functools.partial(
    pl.kernel,
    out_type=jax.ShapeDtypeStruct((NSC, HSZ), jnp.float32),
    mesh=_mesh,
    scratch_types=[
        pltpu.VMEM((CH,), jnp.int32),
        pltpu.VMEM((CH,), jnp.int32),
        pltpu.VMEM((CH,), jnp.float32),
        pltpu.VMEM((1280,), jnp.float32),
        pltpu.VMEM_SHARED((HSZ,), jnp.float32),
    ],
)
def _sc_hist(frd_hbm, dstd_hbm, out_hbm, idx_v, idx2_v, one_v, zb_v, acc_sh):
    cid = lax.axis_index("c")
    sid = lax.axis_index("s")
    base = _wid(cid, sid) * PT

    @pl.loop(0, 1280 // L)
    def _zb(i):
        zb_v[pl.ds(i * L, L)] = jnp.zeros((L,), jnp.float32)

    @pl.loop(0, CH // L)
    def _ones(i):
        one_v[pl.ds(i * L, L)] = jnp.ones((L,), jnp.float32)

    tile_words = HSZ // NSUB

    @pl.loop(0, tile_words // 1280)
    def _z(i):
        pltpu.sync_copy(zb_v, acc_sh.at[pl.ds(sid * tile_words + i * 1280, 1280)])

    plsc.subcore_barrier()

    @pl.loop(0, NCH)
    def _chunk(c):
        off = base + c * CH
        pltpu.sync_copy(frd_hbm.at[pl.ds(off, CH)], idx_v)
        pltpu.sync_copy(dstd_hbm.at[pl.ds(off, CH)], idx2_v)
        pltpu.sync_copy(one_v, acc_sh.at[idx_v], add=True)
        pltpu.sync_copy(one_v, acc_sh.at[idx2_v], add=True)

    plsc.subcore_barrier()
    pltpu.sync_copy(
        acc_sh.at[pl.ds(sid * tile_words, tile_words)],
        out_hbm.at[cid, pl.ds(sid * tile_words, tile_words)],
    )


@functools.partial(
    pl.kernel,
    out_type=(
        jax.ShapeDtypeStruct((EP,), jnp.float32),
        jax.ShapeDtypeStruct((EP,), jnp.float32),
    ),
    mesh=_mesh,
    scratch_types=[
        pltpu.VMEM((2, 3, CH), jnp.int32),
        pltpu.VMEM((2, CH, EMB), jnp.float32),
        pltpu.VMEM((2, CH, EMB), jnp.float32),
        pltpu.VMEM((2, CH), jnp.float32),
        pltpu.VMEM((2, CH), jnp.float32),
        pltpu.VMEM((CH,), jnp.float32),
        pltpu.VMEM((CH,), jnp.float32),
        pltpu.SemaphoreType.DMA,
        pltpu.SemaphoreType.DMA,
    ],
)
def _sc_wcos(xn_hbm, pack_hbm, inv0_hbm, inv1_hbm, a0_hbm, a1_hbm,
             pk, rs, rd, iv0, iv1, ob0, ob1, gsem, isem):
    cid = lax.axis_index("c")
    sid = lax.axis_index("s")
    cbase, nch_t = _tile_chunks(cid, sid)

    @pl.loop(0, nch_t // 2)
    def _outer(m):
        for b in range(2):
            ci = cbase + m * 2 + b
            pltpu.sync_copy(pack_hbm.at[ci], pk.at[b])
        gds, ids = [], []
        for b in range(2):
            gds.append(pltpu.async_copy(xn_hbm.at[pk.at[b, 0]], rs.at[b], gsem))
            gds.append(pltpu.async_copy(xn_hbm.at[pk.at[b, 1]], rd.at[b], gsem))
            ids.append(pltpu.async_copy(inv0_hbm.at[pk.at[b, 2]], iv0.at[b], isem))
            ids.append(pltpu.async_copy(inv1_hbm.at[pk.at[b, 2]], iv1.at[b], isem))
        for b in range(2):
            gds[2 * b].wait()
            gds[2 * b + 1].wait()
            ids[2 * b].wait()
            ids[2 * b + 1].wait()
            off = (cbase + m * 2 + b) * CH

            @pl.loop(0, CH // L)
            def _grp(g, b=b):
                lanes = lax.iota(jnp.int32, L)
                wv = jnp.zeros((L,), jnp.float32)
                for jl in range(L):
                    j = g * L + jl
                    a = rs[b, j, pl.ds(0, L)] * rd[b, j, pl.ds(0, L)]
                    for k in range(1, EMB // L):
                        a = a + rs[b, j, pl.ds(k * L, L)] * rd[b, j, pl.ds(k * L, L)]
                    for sh in (8, 4, 2, 1):
                        a = a + _dyn_gather(a, lanes ^ sh)
                    wv = jnp.where(lanes == jl, a, wv)
                ob0[pl.ds(g * L, L)] = wv * iv0[b, pl.ds(g * L, L)]
                ob1[pl.ds(g * L, L)] = wv * iv1[b, pl.ds(g * L, L)]

            pltpu.sync_copy(ob0, a0_hbm.at[pl.ds(off, CH)])
            pltpu.sync_copy(ob1, a1_hbm.at[pl.ds(off, CH)])


def _make_sc_edge(with_alpha):
    scratch = [
        pltpu.VMEM((NB, 2, CH), jnp.int32),
        pltpu.VMEM((NB, CH, HID), jnp.float32),
        pltpu.VMEM_SHARED((NPAD, HID), jnp.float32),
        pltpu.SemaphoreType.DMA,
        pltpu.SemaphoreType.DMA,
    ]
    if with_alpha:
        scratch.insert(2, pltpu.VMEM((NB, CH), jnp.float32))

    @functools.partial(
        pl.kernel,
        out_type=jax.ShapeDtypeStruct((NSC, NPAD, HID), jnp.float32),
        mesh=_mesh,
        scratch_types=scratch,
    )
    def _sc_edge(tab_hbm, pack_hbm, *rest):
        if with_alpha:
            alpha_hbm, out_hbm, pk, rows, al, acc_sh, gsem, ssem = rest
        else:
            out_hbm, pk, rows, acc_sh, gsem, ssem = rest
        cid = lax.axis_index("c")
        sid = lax.axis_index("s")
        cbase, nch_t = _tile_chunks(cid, sid)

        @pl.loop(0, CH)
        def _zr(i):
            for k in range(HID // L):
                rows[0, i, pl.ds(k * L, L)] = jnp.zeros((L,), jnp.float32)

        for t in range(ROWS_T // CH):
            pltpu.sync_copy(
                rows.at[0],
                acc_sh.at[pl.ds(sid * ROWS_T + t * CH, CH)],
            )
        plsc.subcore_barrier()

        @pl.loop(0, nch_t // NB)
        def _outer(m):
            for b in range(NB):
                ci = cbase + m * NB + b
                pltpu.sync_copy(pack_hbm.at[ci], pk.at[b])
                if with_alpha:
                    pltpu.sync_copy(alpha_hbm.at[pl.ds(ci * CH, CH)], al.at[b])
            gds = []
            for b in range(NB):
                gds.append(
                    pltpu.async_copy(tab_hbm.at[pk.at[b, 0]], rows.at[b], gsem)
                )
            sds = []
            for b in range(NB):
                gds[b].wait()
                if with_alpha:

                    @pl.loop(0, CH // L)
                    def _scale(g, b=b):
                        av = al[b, pl.ds(g * L, L)]
                        for jl in range(L):
                            j = g * L + jl
                            ab = _dyn_gather(av, jnp.full((L,), jl, jnp.int32))
                            for k in range(HID // L):
                                rows[b, j, pl.ds(k * L, L)] = (
                                    rows[b, j, pl.ds(k * L, L)] * ab
                                )

                sds.append(
                    pltpu.async_copy(
                        rows.at[b], acc_sh.at[pk.at[b, 1]], ssem, add=True
                    )
                )
            for b in range(NB):
                sds[b].wait()

        plsc.subcore_barrier()
        for t in range(ROWS_T // CH):
            pltpu.sync_copy(
                acc_sh.at[pl.ds(sid * ROWS_T + t * CH, CH)],
                out_hbm.at[cid, pl.ds(sid * ROWS_T + t * CH, CH)],
            )

    return _sc_edge


_sc_edge_rel = _make_sc_edge(True)
_sc_edge_mp = _make_sc_edge(False)


def _tc_bn(x, gamma, beta):
    def body(x_ref, g_ref, b_ref, xbn_ref, xn_ref):
        xv = x_ref[...]
        m = jnp.mean(xv, axis=0, keepdims=True)
        xc = xv - m
        v = jnp.mean(xc * xc, axis=0, keepdims=True)
        xbn = xc * lax.rsqrt(v + 1e-5) * g_ref[...] + b_ref[...]
        s = jnp.sum(xbn * xbn, axis=1, keepdims=True)
        xbn_ref[...] = xbn
        xn_ref[...] = xbn * lax.rsqrt(s)

    return pl.pallas_call(
        body,
        out_shape=[
            jax.ShapeDtypeStruct((N, EMB), jnp.float32),
            jax.ShapeDtypeStruct((N, EMB), jnp.float32),
        ],
    )(x, gamma, beta)


def _tc_ytab(x, relWi, relbi):
    def body(x_ref, w_ref, b_ref, o_ref):
        o_ref[0] = (
            jnp.dot(x_ref[...], w_ref[0], preferred_element_type=jnp.float32)
            + b_ref[0]
        )

    return pl.pallas_call(
        body,
        grid=(NREL,),
        in_specs=[
            pl.BlockSpec((N, EMB), lambda r: (0, 0)),
            pl.BlockSpec((1, EMB, HID), lambda r: (r, 0, 0)),
            pl.BlockSpec((1, 1, HID), lambda r: (r, 0, 0)),
        ],
        out_specs=pl.BlockSpec((1, N, HID), lambda r: (r, 0, 0)),
        out_shape=jax.ShapeDtypeStruct((NREL, N, HID), jnp.float32),
    )(x, relWi, relbi.reshape(NREL, 1, HID))


def _tc_invtab(hist, normc):
    def body(h_ref, nc_ref, ir_ref, id_ref):
        cnt = h_ref[0] + h_ref[1]
        crel = 1.0 / jnp.maximum(cnt[:RELSZ], 1.0)
        for i in range(NL):
            fac = jnp.concatenate(
                [jnp.full((N,), 1.0 / nc_ref[i, r], jnp.float32) for r in range(NREL)]
            )
            ir_ref[i] = fac * crel
        id_ref[...] = 1.0 / jnp.maximum(cnt[DEG_OFF : DEG_OFF + N], 1.0)

    return pl.pallas_call(
        body,
        in_specs=[
            pl.BlockSpec(memory_space=pltpu.VMEM),
            pl.BlockSpec(memory_space=pltpu.SMEM),
        ],
        out_shape=[
            jax.ShapeDtypeStruct((NL, RELSZ), jnp.float32),
            jax.ShapeDtypeStruct((N,), jnp.float32),
        ],
    )(hist, normc)


def _tc_relucomb(accp):
    def body(a_ref, o_ref):
        o_ref[...] = jnp.maximum(a_ref[0] + a_ref[1], 0.0)

    return pl.pallas_call(
        body,
        grid=(1,),
        in_specs=[pl.BlockSpec((NSC, N, HID), lambda i: (0, 0, 0))],
        out_specs=pl.BlockSpec((N, HID), lambda i: (0, 0)),
        out_shape=jax.ShapeDtypeStruct((N, HID), jnp.float32),
    )(accp)


def _tc_zs(h, W, b, Ws, bs):
    def body(h_ref, w_ref, b_ref, ws_ref, bs_ref, z_ref, s_ref):
        hv = h_ref[...]
        z_ref[...] = (
            jnp.dot(hv, w_ref[...], preferred_element_type=jnp.float32) + b_ref[...]
        )
        s_ref[...] = (
            jnp.dot(hv, ws_ref[...], preferred_element_type=jnp.float32) + bs_ref[...]
        )

    return pl.pallas_call(
        body,
        out_shape=[
            jax.ShapeDtypeStruct((N, HID), jnp.float32),
            jax.ShapeDtypeStruct((N, HID), jnp.float32),
        ],
    )(h, W, b, Ws, bs)


def _tc_comb2(aggp, inv_deg2, s, xprev):
    def body(g_ref, id_ref, s_ref, xp_ref, o_ref):
        aggr = (g_ref[0] + g_ref[1]) * id_ref[...]
        o_ref[...] = jnp.maximum(aggr + s_ref[...], 0.0) + xp_ref[...]

    return pl.pallas_call(
        body,
        grid=(1,),
        in_specs=[
            pl.BlockSpec((NSC, N, HID), lambda i: (0, 0, 0)),
            pl.BlockSpec((N, 1), lambda i: (0, 0)),
            pl.BlockSpec((N, HID), lambda i: (0, 0)),
            pl.BlockSpec((N, HID), lambda i: (0, 0)),
        ],
        out_specs=pl.BlockSpec((N, HID), lambda i: (0, 0)),
        out_shape=jax.ShapeDtypeStruct((N, HID), jnp.float32),
    )(aggp, inv_deg2, s, xprev)


def _tc_batchmean(x):
    def body(x_ref, o_ref):
        o_ref[...] = jnp.mean(x_ref[...], axis=1, keepdims=True)

    npb = N // BATCH
    out = pl.pallas_call(
        body,
        grid=(BATCH,),
        in_specs=[pl.BlockSpec((1, npb, EMB), lambda b: (b, 0, 0))],
        out_specs=pl.BlockSpec((1, 1, EMB), lambda b: (b, 0, 0)),
        out_shape=jax.ShapeDtypeStruct((BATCH, 1, EMB), jnp.float32),
    )(x.reshape(BATCH, npb, EMB))
    return out.reshape(BATCH, EMB)


def kernel(x, edge_index, edge_type, batch_size, bn_gamma, bn_beta, relW, relb,
           normc, mpW, mpb, selfW, selfb):
    E = edge_index.shape[1]
    pad = EP - E

    xbn, xn = _tc_bn(x, bn_gamma, bn_beta)

    srcp = jnp.concatenate([edge_index[0], jnp.zeros((pad,), jnp.int32)])
    dstp = jnp.concatenate([edge_index[1], jnp.full((pad,), N, jnp.int32)])
    etp = jnp.concatenate([edge_type, jnp.zeros((pad,), jnp.int32)])
    gi_rel = etp * N + srcp
    frd_iv = etp * N + dstp
    frd_hist = jnp.concatenate(
        [frd_iv[:E], jnp.full((pad,), PAD_RD, jnp.int32)]
    )
    dst_deg = dstp + DEG_OFF

    pack_rel = jnp.stack(
        [gi_rel.reshape(TOTCH, CH), dstp.reshape(TOTCH, CH)], axis=1
    )
    pack_mp = jnp.stack(
        [srcp.reshape(TOTCH, CH), dstp.reshape(TOTCH, CH)], axis=1
    )
    pack_w = jnp.stack(
        [
            srcp.reshape(TOTCH, CH),
            dstp.reshape(TOTCH, CH),
            frd_iv.reshape(TOTCH, CH),
        ],
        axis=1,
    )

    hist = _sc_hist(frd_hist, dst_deg)
    inv_rel, inv_deg = _tc_invtab(hist, normc)
    alphas = _sc_wcos(xn, pack_w, inv_rel[0], inv_rel[1])
    inv_deg2 = inv_deg.reshape(N, 1)

    xcur = xbn
    for i in range(NL):
        ytab = _tc_ytab(xcur, relW[i], relb[i]).reshape(NREL * N, HID)
        accp = _sc_edge_rel(ytab, pack_rel, alphas[i])
        h = _tc_relucomb(accp)
        z, s = _tc_zs(h, mpW[i], mpb[i], selfW[i], selfb[i])
        aggp = _sc_edge_mp(z, pack_mp)
        xcur = _tc_comb2(aggp, inv_deg2, s, xcur)

    return _tc_batchmean(xcur)


# speedup vs baseline: 1.1063x; 1.1063x over previous
import functools

import jax
import jax.numpy as jnp
from jax import lax
from jax.experimental import pallas as pl
from jax.experimental.pallas import tpu as pltpu
from jax.experimental.pallas import tpu_sc as plsc

N = 10000
EMB = 128
HID = 128
NREL = 3
NL = 2
BATCH = 100

NSC = 2
NSUB = 16
NW = NSC * NSUB
L = 16

CH = 128
PT = 5120
EP = NW * PT
NCH = PT // CH
TOTCH = EP // CH
NB = 2

RELSZ = NREL * N
PAD_RD = RELSZ
DEG_OFF = 30720
HSZ = 40960
NPAD = 10240
ROWS_T = NPAD // NSUB

F0 = 60
F1 = (2 * NCH) - F0

_mesh = plsc.VectorSubcoreMesh(core_axis_name="c", subcore_axis_name="s")


def _tile_chunks(cid, sid):
    cbase = jnp.where(cid == 0, sid * F0, NSUB * F0 + sid * F1)
    nch_t = jnp.where(cid == 0, F0, F1)
    return cbase, nch_t


def _wid(cid, sid):
    return sid * NSC + cid


def _dyn_gather(a, idx):
    dn = lax.GatherDimensionNumbers(
        offset_dims=(), collapsed_slice_dims=(0,), start_index_map=(0,)
    )
    return lax.gather(
        a, idx[:, None], dn, slice_sizes=(1,),
        mode=lax.GatherScatterMode.PROMISE_IN_BOUNDS,
    )


@functools.partial(
    pl.kernel,
    out_type=jax.ShapeDtypeStruct((NSC, HSZ), jnp.float32),
    mesh=_mesh,
    scratch_types=[
        pltpu.VMEM((CH,), jnp.int32),
        pltpu.VMEM((CH,), jnp.int32),
        pltpu.VMEM((CH,), jnp.float32),
        pltpu.VMEM((1280,), jnp.float32),
        pltpu.VMEM_SHARED((HSZ,), jnp.float32),
    ],
)
def _sc_hist(frd_hbm, dstd_hbm, out_hbm, idx_v, idx2_v, one_v, zb_v, acc_sh):
    cid = lax.axis_index("c")
    sid = lax.axis_index("s")
    base = _wid(cid, sid) * PT

    @pl.loop(0, 1280 // L)
    def _zb(i):
        zb_v[pl.ds(i * L, L)] = jnp.zeros((L,), jnp.float32)

    @pl.loop(0, CH // L)
    def _ones(i):
        one_v[pl.ds(i * L, L)] = jnp.ones((L,), jnp.float32)

    tile_words = HSZ // NSUB

    @pl.loop(0, tile_words // 1280)
    def _z(i):
        pltpu.sync_copy(zb_v, acc_sh.at[pl.ds(sid * tile_words + i * 1280, 1280)])

    plsc.subcore_barrier()

    @pl.loop(0, NCH)
    def _chunk(c):
        off = base + c * CH
        pltpu.sync_copy(frd_hbm.at[pl.ds(off, CH)], idx_v)
        pltpu.sync_copy(dstd_hbm.at[pl.ds(off, CH)], idx2_v)
        pltpu.sync_copy(one_v, acc_sh.at[idx_v], add=True)
        pltpu.sync_copy(one_v, acc_sh.at[idx2_v], add=True)

    plsc.subcore_barrier()
    pltpu.sync_copy(
        acc_sh.at[pl.ds(sid * tile_words, tile_words)],
        out_hbm.at[cid, pl.ds(sid * tile_words, tile_words)],
    )


@functools.partial(
    pl.kernel,
    out_type=(
        jax.ShapeDtypeStruct((EP,), jnp.float32),
        jax.ShapeDtypeStruct((EP,), jnp.float32),
    ),
    mesh=_mesh,
    scratch_types=[
        pltpu.VMEM((2, 3, CH), jnp.int32),
        pltpu.VMEM((2, CH, EMB), jnp.float32),
        pltpu.VMEM((2, CH, EMB), jnp.float32),
        pltpu.VMEM((2, CH), jnp.float32),
        pltpu.VMEM((2, CH), jnp.float32),
        pltpu.VMEM((CH,), jnp.float32),
        pltpu.VMEM((CH,), jnp.float32),
        pltpu.SemaphoreType.DMA,
        pltpu.SemaphoreType.DMA,
    ],
)
def _sc_wcos(xn_hbm, pack_hbm, inv0_hbm, inv1_hbm, a0_hbm, a1_hbm,
             pk, rs, rd, iv0, iv1, ob0, ob1, gsem, isem):
    cid = lax.axis_index("c")
    sid = lax.axis_index("s")
    cbase, nch_t = _tile_chunks(cid, sid)

    @pl.loop(0, nch_t // 2)
    def _outer(m):
        for b in range(2):
            ci = cbase + m * 2 + b
            pltpu.sync_copy(pack_hbm.at[ci], pk.at[b])
        gds, ids = [], []
        for b in range(2):
            gds.append(pltpu.async_copy(xn_hbm.at[pk.at[b, 0]], rs.at[b], gsem))
            gds.append(pltpu.async_copy(xn_hbm.at[pk.at[b, 1]], rd.at[b], gsem))
            ids.append(pltpu.async_copy(inv0_hbm.at[pk.at[b, 2]], iv0.at[b], isem))
            ids.append(pltpu.async_copy(inv1_hbm.at[pk.at[b, 2]], iv1.at[b], isem))
        for b in range(2):
            gds[2 * b].wait()
            gds[2 * b + 1].wait()
            ids[2 * b].wait()
            ids[2 * b + 1].wait()
            off = (cbase + m * 2 + b) * CH

            @pl.loop(0, CH // L)
            def _grp(g, b=b):
                lanes = lax.iota(jnp.int32, L)
                wv = jnp.zeros((L,), jnp.float32)
                for jl in range(L):
                    j = g * L + jl
                    a = rs[b, j, pl.ds(0, L)] * rd[b, j, pl.ds(0, L)]
                    for k in range(1, EMB // L):
                        a = a + rs[b, j, pl.ds(k * L, L)] * rd[b, j, pl.ds(k * L, L)]
                    for sh in (8, 4, 2, 1):
                        a = a + _dyn_gather(a, lanes ^ sh)
                    wv = jnp.where(lanes == jl, a, wv)
                ob0[pl.ds(g * L, L)] = wv * iv0[b, pl.ds(g * L, L)]
                ob1[pl.ds(g * L, L)] = wv * iv1[b, pl.ds(g * L, L)]

            pltpu.sync_copy(ob0, a0_hbm.at[pl.ds(off, CH)])
            pltpu.sync_copy(ob1, a1_hbm.at[pl.ds(off, CH)])


def _make_sc_edge(with_alpha):
    scratch = [
        pltpu.VMEM((NB, 2, CH), jnp.int32),
        pltpu.VMEM((NB, CH, HID), jnp.float32),
        pltpu.VMEM_SHARED((NPAD, HID), jnp.float32),
        pltpu.SemaphoreType.DMA,
        pltpu.SemaphoreType.DMA,
    ]
    if with_alpha:
        scratch.insert(2, pltpu.VMEM((NB, CH), jnp.float32))

    @functools.partial(
        pl.kernel,
        out_type=jax.ShapeDtypeStruct((NSC, NPAD, HID), jnp.float32),
        mesh=_mesh,
        scratch_types=scratch,
    )
    def _sc_edge(tab_hbm, pack_hbm, *rest):
        if with_alpha:
            alpha_hbm, out_hbm, pk, rows, al, acc_sh, gsem, ssem = rest
        else:
            out_hbm, pk, rows, acc_sh, gsem, ssem = rest
        cid = lax.axis_index("c")
        sid = lax.axis_index("s")
        cbase, nch_t = _tile_chunks(cid, sid)

        @pl.loop(0, CH)
        def _zr(i):
            for k in range(HID // L):
                rows[0, i, pl.ds(k * L, L)] = jnp.zeros((L,), jnp.float32)

        for t in range(ROWS_T // CH):
            pltpu.sync_copy(
                rows.at[0],
                acc_sh.at[pl.ds(sid * ROWS_T + t * CH, CH)],
            )
        plsc.subcore_barrier()

        @pl.loop(0, nch_t // NB)
        def _outer(m):
            for b in range(NB):
                ci = cbase + m * NB + b
                pltpu.sync_copy(pack_hbm.at[ci], pk.at[b])
                if with_alpha:
                    pltpu.sync_copy(alpha_hbm.at[pl.ds(ci * CH, CH)], al.at[b])
            gds = []
            for b in range(NB):
                gds.append(
                    pltpu.async_copy(tab_hbm.at[pk.at[b, 0]], rows.at[b], gsem)
                )
            sds = []
            for b in range(NB):
                gds[b].wait()
                if with_alpha:

                    @pl.loop(0, CH // L)
                    def _scale(g, b=b):
                        av = al[b, pl.ds(g * L, L)]
                        for jl in range(L):
                            j = g * L + jl
                            ab = _dyn_gather(av, jnp.full((L,), jl, jnp.int32))
                            for k in range(HID // L):
                                rows[b, j, pl.ds(k * L, L)] = (
                                    rows[b, j, pl.ds(k * L, L)] * ab
                                )

                sds.append(
                    pltpu.async_copy(
                        rows.at[b], acc_sh.at[pk.at[b, 1]], ssem, add=True
                    )
                )
            for b in range(NB):
                sds[b].wait()

        plsc.subcore_barrier()
        for t in range(ROWS_T // CH):
            pltpu.sync_copy(
                acc_sh.at[pl.ds(sid * ROWS_T + t * CH, CH)],
                out_hbm.at[cid, pl.ds(sid * ROWS_T + t * CH, CH)],
            )

    return _sc_edge


_sc_edge_rel = _make_sc_edge(True)
_sc_edge_mp = _make_sc_edge(False)


def _tc_bn(x, gamma, beta):
    def body(x_ref, g_ref, b_ref, xbn_ref, xn_ref):
        xv = x_ref[...]
        m = jnp.mean(xv, axis=0, keepdims=True)
        xc = xv - m
        v = jnp.mean(xc * xc, axis=0, keepdims=True)
        xbn = xc * lax.rsqrt(v + 1e-5) * g_ref[...] + b_ref[...]
        s = jnp.sum(xbn * xbn, axis=1, keepdims=True)
        xbn_ref[...] = xbn
        xn_ref[...] = xbn * lax.rsqrt(s)

    return pl.pallas_call(
        body,
        out_shape=[
            jax.ShapeDtypeStruct((N, EMB), jnp.float32),
            jax.ShapeDtypeStruct((N, EMB), jnp.float32),
        ],
    )(x, gamma, beta)


def _tc_ytab(x, relWi, relbi):
    def body(x_ref, w_ref, b_ref, o_ref):
        o_ref[0] = (
            jnp.dot(x_ref[...], w_ref[0], preferred_element_type=jnp.float32)
            + b_ref[0]
        )

    return pl.pallas_call(
        body,
        grid=(NREL,),
        in_specs=[
            pl.BlockSpec((N, EMB), lambda r: (0, 0)),
            pl.BlockSpec((1, EMB, HID), lambda r: (r, 0, 0)),
            pl.BlockSpec((1, 1, HID), lambda r: (r, 0, 0)),
        ],
        out_specs=pl.BlockSpec((1, N, HID), lambda r: (r, 0, 0)),
        out_shape=jax.ShapeDtypeStruct((NREL, N, HID), jnp.float32),
    )(x, relWi, relbi.reshape(NREL, 1, HID))


def _tc_invtab(hist, normc):
    def body(h_ref, nc_ref, ir_ref, id_ref):
        cnt = h_ref[0] + h_ref[1]
        crel = 1.0 / jnp.maximum(cnt[:RELSZ], 1.0)
        for i in range(NL):
            fac = jnp.concatenate(
                [jnp.full((N,), 1.0 / nc_ref[i, r], jnp.float32) for r in range(NREL)]
            )
            ir_ref[i] = fac * crel
        id_ref[...] = 1.0 / jnp.maximum(cnt[DEG_OFF : DEG_OFF + N], 1.0)

    return pl.pallas_call(
        body,
        in_specs=[
            pl.BlockSpec(memory_space=pltpu.VMEM),
            pl.BlockSpec(memory_space=pltpu.SMEM),
        ],
        out_shape=[
            jax.ShapeDtypeStruct((NL, RELSZ), jnp.float32),
            jax.ShapeDtypeStruct((N,), jnp.float32),
        ],
    )(hist, normc)


def _tc_relucomb(accp):
    def body(a_ref, o_ref):
        o_ref[...] = jnp.maximum(a_ref[0] + a_ref[1], 0.0)

    return pl.pallas_call(
        body,
        grid=(1,),
        in_specs=[pl.BlockSpec((NSC, N, HID), lambda i: (0, 0, 0))],
        out_specs=pl.BlockSpec((N, HID), lambda i: (0, 0)),
        out_shape=jax.ShapeDtypeStruct((N, HID), jnp.float32),
    )(accp)


def _tc_zs(h, W, b, Ws, bs):
    def body(h_ref, w_ref, b_ref, ws_ref, bs_ref, z_ref, s_ref):
        hv = h_ref[...]
        z_ref[...] = (
            jnp.dot(hv, w_ref[...], preferred_element_type=jnp.float32) + b_ref[...]
        )
        s_ref[...] = (
            jnp.dot(hv, ws_ref[...], preferred_element_type=jnp.float32) + bs_ref[...]
        )

    return pl.pallas_call(
        body,
        out_shape=[
            jax.ShapeDtypeStruct((N, HID), jnp.float32),
            jax.ShapeDtypeStruct((N, HID), jnp.float32),
        ],
    )(h, W, b, Ws, bs)


def _tc_comb2(aggp, inv_deg2, s, xprev):
    def body(g_ref, id_ref, s_ref, xp_ref, o_ref):
        aggr = (g_ref[0] + g_ref[1]) * id_ref[...]
        o_ref[...] = jnp.maximum(aggr + s_ref[...], 0.0) + xp_ref[...]

    return pl.pallas_call(
        body,
        grid=(1,),
        in_specs=[
            pl.BlockSpec((NSC, N, HID), lambda i: (0, 0, 0)),
            pl.BlockSpec((N, 1), lambda i: (0, 0)),
            pl.BlockSpec((N, HID), lambda i: (0, 0)),
            pl.BlockSpec((N, HID), lambda i: (0, 0)),
        ],
        out_specs=pl.BlockSpec((N, HID), lambda i: (0, 0)),
        out_shape=jax.ShapeDtypeStruct((N, HID), jnp.float32),
    )(aggp, inv_deg2, s, xprev)


def _tc_batchmean(x):
    def body(x_ref, o_ref):
        o_ref[...] = jnp.mean(x_ref[...], axis=1, keepdims=True)

    npb = N // BATCH
    out = pl.pallas_call(
        body,
        grid=(BATCH,),
        in_specs=[pl.BlockSpec((1, npb, EMB), lambda b: (b, 0, 0))],
        out_specs=pl.BlockSpec((1, 1, EMB), lambda b: (b, 0, 0)),
        out_shape=jax.ShapeDtypeStruct((BATCH, 1, EMB), jnp.float32),
    )(x.reshape(BATCH, npb, EMB))
    return out.reshape(BATCH, EMB)


def kernel(x, edge_index, edge_type, batch_size, bn_gamma, bn_beta, relW, relb,
           normc, mpW, mpb, selfW, selfb):
    E = edge_index.shape[1]
    pad = EP - E

    xbn, xn = _tc_bn(x, bn_gamma, bn_beta)

    srcp = jnp.concatenate([edge_index[0], jnp.zeros((pad,), jnp.int32)])
    dstp = jnp.concatenate([edge_index[1], jnp.full((pad,), N, jnp.int32)])
    etp = jnp.concatenate([edge_type, jnp.zeros((pad,), jnp.int32)])
    gi_rel = etp * N + srcp
    frd_iv = etp * N + dstp
    frd_hist = jnp.concatenate(
        [frd_iv[:E], jnp.full((pad,), PAD_RD, jnp.int32)]
    )
    dst_deg = dstp + DEG_OFF

    pack_rel = jnp.stack(
        [gi_rel.reshape(TOTCH, CH), dstp.reshape(TOTCH, CH)], axis=1
    )
    pack_mp = jnp.stack(
        [srcp.reshape(TOTCH, CH), dstp.reshape(TOTCH, CH)], axis=1
    )
    pack_w = jnp.stack(
        [
            srcp.reshape(TOTCH, CH),
            dstp.reshape(TOTCH, CH),
            frd_iv.reshape(TOTCH, CH),
        ],
        axis=1,
    )

    hist = _sc_hist(frd_hist, dst_deg)
    inv_rel, inv_deg = _tc_invtab(hist, normc)
    alphas = _sc_wcos(xn, pack_w, inv_rel[0], inv_rel[1])
    inv_deg2 = inv_deg.reshape(N, 1)

    xcur = xbn
    for i in range(NL):
        ytab = _tc_ytab(xcur, relW[i], relb[i]).reshape(NREL * N, HID)
        accp = _sc_edge_rel(ytab, pack_rel, alphas[i])
        h = _tc_relucomb(accp)
        z, s = _tc_zs(h, mpW[i], mpb[i], selfW[i], selfb[i])
        aggp = _sc_edge_mp(z, pack_mp)
        xcur = _tc_comb2(aggp, inv_deg2, s, xcur)

    return _tc_batchmean(xcur)


# speedup vs baseline: 1.1442x; 1.0342x over previous
import functools

import jax
import jax.numpy as jnp
from jax import lax
from jax.experimental import pallas as pl
from jax.experimental.pallas import tpu as pltpu
from jax.experimental.pallas import tpu_sc as plsc

N = 10000
EMB = 128
HID = 128
NREL = 3
NL = 2
BATCH = 100

NSC = 2
NSUB = 16
NW = NSC * NSUB
L = 16

CH = 128
PT = 5120
EP = NW * PT
NCH = PT // CH
TOTCH = EP // CH
NB = 2

RELSZ = NREL * N
PAD_RD = RELSZ
DEG_OFF = 30720
HSZ = 40960
NPAD = 10240
ROWS_T = NPAD // NSUB

F0 = 64
F1 = (2 * NCH) - F0

_mesh = plsc.VectorSubcoreMesh(core_axis_name="c", subcore_axis_name="s")


def _tile_chunks(cid, sid):
    cbase = jnp.where(cid == 0, sid * F0, NSUB * F0 + sid * F1)
    nch_t = jnp.where(cid == 0, F0, F1)
    return cbase, nch_t


def _wid(cid, sid):
    return sid * NSC + cid


def _dyn_gather(a, idx):
    dn = lax.GatherDimensionNumbers(
        offset_dims=(), collapsed_slice_dims=(0,), start_index_map=(0,)
    )
    return lax.gather(
        a, idx[:, None], dn, slice_sizes=(1,),
        mode=lax.GatherScatterMode.PROMISE_IN_BOUNDS,
    )


@functools.partial(
    pl.kernel,
    out_type=jax.ShapeDtypeStruct((NSC, HSZ), jnp.float32),
    mesh=_mesh,
    scratch_types=[
        pltpu.VMEM((CH,), jnp.int32),
        pltpu.VMEM((CH,), jnp.int32),
        pltpu.VMEM((CH,), jnp.float32),
        pltpu.VMEM((1280,), jnp.float32),
        pltpu.VMEM_SHARED((HSZ,), jnp.float32),
    ],
)
def _sc_hist(frd_hbm, dstd_hbm, out_hbm, idx_v, idx2_v, one_v, zb_v, acc_sh):
    cid = lax.axis_index("c")
    sid = lax.axis_index("s")
    base = _wid(cid, sid) * PT

    @pl.loop(0, 1280 // L)
    def _zb(i):
        zb_v[pl.ds(i * L, L)] = jnp.zeros((L,), jnp.float32)

    @pl.loop(0, CH // L)
    def _ones(i):
        one_v[pl.ds(i * L, L)] = jnp.ones((L,), jnp.float32)

    tile_words = HSZ // NSUB

    @pl.loop(0, tile_words // 1280)
    def _z(i):
        pltpu.sync_copy(zb_v, acc_sh.at[pl.ds(sid * tile_words + i * 1280, 1280)])

    plsc.subcore_barrier()

    @pl.loop(0, NCH)
    def _chunk(c):
        off = base + c * CH
        pltpu.sync_copy(frd_hbm.at[pl.ds(off, CH)], idx_v)
        pltpu.sync_copy(dstd_hbm.at[pl.ds(off, CH)], idx2_v)
        pltpu.sync_copy(one_v, acc_sh.at[idx_v], add=True)
        pltpu.sync_copy(one_v, acc_sh.at[idx2_v], add=True)

    plsc.subcore_barrier()
    pltpu.sync_copy(
        acc_sh.at[pl.ds(sid * tile_words, tile_words)],
        out_hbm.at[cid, pl.ds(sid * tile_words, tile_words)],
    )


@functools.partial(
    pl.kernel,
    out_type=(
        jax.ShapeDtypeStruct((EP,), jnp.float32),
        jax.ShapeDtypeStruct((EP,), jnp.float32),
    ),
    mesh=_mesh,
    scratch_types=[
        pltpu.VMEM((2, 3, CH), jnp.int32),
        pltpu.VMEM((2, CH, EMB), jnp.float32),
        pltpu.VMEM((2, CH, EMB), jnp.float32),
        pltpu.VMEM((2, CH), jnp.float32),
        pltpu.VMEM((2, CH), jnp.float32),
        pltpu.VMEM((CH,), jnp.float32),
        pltpu.VMEM((CH,), jnp.float32),
        pltpu.SemaphoreType.DMA,
        pltpu.SemaphoreType.DMA,
    ],
)
def _sc_wcos(xn_hbm, pack_hbm, inv0_hbm, inv1_hbm, a0_hbm, a1_hbm,
             pk, rs, rd, iv0, iv1, ob0, ob1, gsem, isem):
    cid = lax.axis_index("c")
    sid = lax.axis_index("s")
    cbase, nch_t = _tile_chunks(cid, sid)

    @pl.loop(0, nch_t // 2)
    def _outer(m):
        for b in range(2):
            ci = cbase + m * 2 + b
            pltpu.sync_copy(pack_hbm.at[ci], pk.at[b])
        gds, ids = [], []
        for b in range(2):
            gds.append(pltpu.async_copy(xn_hbm.at[pk.at[b, 0]], rs.at[b], gsem))
            gds.append(pltpu.async_copy(xn_hbm.at[pk.at[b, 1]], rd.at[b], gsem))
            ids.append(pltpu.async_copy(inv0_hbm.at[pk.at[b, 2]], iv0.at[b], isem))
            ids.append(pltpu.async_copy(inv1_hbm.at[pk.at[b, 2]], iv1.at[b], isem))
        for b in range(2):
            gds[2 * b].wait()
            gds[2 * b + 1].wait()
            ids[2 * b].wait()
            ids[2 * b + 1].wait()
            off = (cbase + m * 2 + b) * CH

            @pl.loop(0, CH // L)
            def _grp(g, b=b):
                lanes = lax.iota(jnp.int32, L)
                wv = jnp.zeros((L,), jnp.float32)
                for jl in range(L):
                    j = g * L + jl
                    a = rs[b, j, pl.ds(0, L)] * rd[b, j, pl.ds(0, L)]
                    for k in range(1, EMB // L):
                        a = a + rs[b, j, pl.ds(k * L, L)] * rd[b, j, pl.ds(k * L, L)]
                    for sh in (8, 4, 2, 1):
                        a = a + _dyn_gather(a, lanes ^ sh)
                    wv = jnp.where(lanes == jl, a, wv)
                ob0[pl.ds(g * L, L)] = wv * iv0[b, pl.ds(g * L, L)]
                ob1[pl.ds(g * L, L)] = wv * iv1[b, pl.ds(g * L, L)]

            pltpu.sync_copy(ob0, a0_hbm.at[pl.ds(off, CH)])
            pltpu.sync_copy(ob1, a1_hbm.at[pl.ds(off, CH)])


def _make_sc_edge(with_alpha):
    scratch = [
        pltpu.VMEM((NB, 2, CH), jnp.int32),
        pltpu.VMEM((NB, CH, HID), jnp.float32),
        pltpu.VMEM_SHARED((NPAD, HID), jnp.float32),
        pltpu.SemaphoreType.DMA,
        pltpu.SemaphoreType.DMA,
    ]
    if with_alpha:
        scratch.insert(2, pltpu.VMEM((NB, CH), jnp.float32))

    @functools.partial(
        pl.kernel,
        out_type=jax.ShapeDtypeStruct((NSC, NPAD, HID), jnp.float32),
        mesh=_mesh,
        scratch_types=scratch,
    )
    def _sc_edge(tab_hbm, pack_hbm, *rest):
        if with_alpha:
            alpha_hbm, out_hbm, pk, rows, al, acc_sh, gsem, ssem = rest
        else:
            out_hbm, pk, rows, acc_sh, gsem, ssem = rest
        cid = lax.axis_index("c")
        sid = lax.axis_index("s")
        cbase, nch_t = _tile_chunks(cid, sid)

        @pl.loop(0, CH)
        def _zr(i):
            for k in range(HID // L):
                rows[0, i, pl.ds(k * L, L)] = jnp.zeros((L,), jnp.float32)

        for t in range(ROWS_T // CH):
            pltpu.sync_copy(
                rows.at[0],
                acc_sh.at[pl.ds(sid * ROWS_T + t * CH, CH)],
            )
        plsc.subcore_barrier()

        @pl.loop(0, nch_t // NB)
        def _outer(m):
            for b in range(NB):
                ci = cbase + m * NB + b
                pltpu.sync_copy(pack_hbm.at[ci], pk.at[b])
                if with_alpha:
                    pltpu.sync_copy(alpha_hbm.at[pl.ds(ci * CH, CH)], al.at[b])
            gds = []
            for b in range(NB):
                gds.append(
                    pltpu.async_copy(tab_hbm.at[pk.at[b, 0]], rows.at[b], gsem)
                )
            sds = []
            for b in range(NB):
                gds[b].wait()
                if with_alpha:

                    @pl.loop(0, CH // L)
                    def _scale(g, b=b):
                        av = al[b, pl.ds(g * L, L)]
                        for jl in range(L):
                            j = g * L + jl
                            ab = _dyn_gather(av, jnp.full((L,), jl, jnp.int32))
                            for k in range(HID // L):
                                rows[b, j, pl.ds(k * L, L)] = (
                                    rows[b, j, pl.ds(k * L, L)] * ab
                                )

                sds.append(
                    pltpu.async_copy(
                        rows.at[b], acc_sh.at[pk.at[b, 1]], ssem, add=True
                    )
                )
            for b in range(NB):
                sds[b].wait()

        plsc.subcore_barrier()
        for t in range(ROWS_T // CH):
            pltpu.sync_copy(
                acc_sh.at[pl.ds(sid * ROWS_T + t * CH, CH)],
                out_hbm.at[cid, pl.ds(sid * ROWS_T + t * CH, CH)],
            )

    return _sc_edge


_sc_edge_rel = _make_sc_edge(True)
_sc_edge_mp = _make_sc_edge(False)


def _tc_bn(x, gamma, beta):
    def body(x_ref, g_ref, b_ref, xbn_ref, xn_ref):
        xv = x_ref[...]
        m = jnp.mean(xv, axis=0, keepdims=True)
        xc = xv - m
        v = jnp.mean(xc * xc, axis=0, keepdims=True)
        xbn = xc * lax.rsqrt(v + 1e-5) * g_ref[...] + b_ref[...]
        s = jnp.sum(xbn * xbn, axis=1, keepdims=True)
        xbn_ref[...] = xbn
        xn_ref[...] = xbn * lax.rsqrt(s)

    return pl.pallas_call(
        body,
        out_shape=[
            jax.ShapeDtypeStruct((N, EMB), jnp.float32),
            jax.ShapeDtypeStruct((N, EMB), jnp.float32),
        ],
    )(x, gamma, beta)


def _tc_ytab(x, relWi, relbi):
    def body(x_ref, w_ref, b_ref, o_ref):
        o_ref[0] = (
            jnp.dot(x_ref[...], w_ref[0], preferred_element_type=jnp.float32)
            + b_ref[0]
        )

    return pl.pallas_call(
        body,
        grid=(NREL,),
        in_specs=[
            pl.BlockSpec((N, EMB), lambda r: (0, 0)),
            pl.BlockSpec((1, EMB, HID), lambda r: (r, 0, 0)),
            pl.BlockSpec((1, 1, HID), lambda r: (r, 0, 0)),
        ],
        out_specs=pl.BlockSpec((1, N, HID), lambda r: (r, 0, 0)),
        out_shape=jax.ShapeDtypeStruct((NREL, N, HID), jnp.float32),
    )(x, relWi, relbi.reshape(NREL, 1, HID))


def _tc_invtab(hist, normc):
    def body(h_ref, nc_ref, ir_ref, id_ref):
        cnt = h_ref[0] + h_ref[1]
        crel = 1.0 / jnp.maximum(cnt[:RELSZ], 1.0)
        for i in range(NL):
            fac = jnp.concatenate(
                [jnp.full((N,), 1.0 / nc_ref[i, r], jnp.float32) for r in range(NREL)]
            )
            ir_ref[i] = fac * crel
        id_ref[...] = 1.0 / jnp.maximum(cnt[DEG_OFF : DEG_OFF + N], 1.0)

    return pl.pallas_call(
        body,
        in_specs=[
            pl.BlockSpec(memory_space=pltpu.VMEM),
            pl.BlockSpec(memory_space=pltpu.SMEM),
        ],
        out_shape=[
            jax.ShapeDtypeStruct((NL, RELSZ), jnp.float32),
            jax.ShapeDtypeStruct((N,), jnp.float32),
        ],
    )(hist, normc)


def _tc_relucomb(accp):
    def body(a_ref, o_ref):
        o_ref[...] = jnp.maximum(a_ref[0] + a_ref[1], 0.0)

    return pl.pallas_call(
        body,
        grid=(1,),
        in_specs=[pl.BlockSpec((NSC, N, HID), lambda i: (0, 0, 0))],
        out_specs=pl.BlockSpec((N, HID), lambda i: (0, 0)),
        out_shape=jax.ShapeDtypeStruct((N, HID), jnp.float32),
    )(accp)


def _tc_zs(h, W, b, Ws, bs):
    def body(h_ref, w_ref, b_ref, ws_ref, bs_ref, z_ref, s_ref):
        hv = h_ref[...]
        z_ref[...] = (
            jnp.dot(hv, w_ref[...], preferred_element_type=jnp.float32) + b_ref[...]
        )
        s_ref[...] = (
            jnp.dot(hv, ws_ref[...], preferred_element_type=jnp.float32) + bs_ref[...]
        )

    return pl.pallas_call(
        body,
        out_shape=[
            jax.ShapeDtypeStruct((N, HID), jnp.float32),
            jax.ShapeDtypeStruct((N, HID), jnp.float32),
        ],
    )(h, W, b, Ws, bs)


def _tc_comb2(aggp, inv_deg2, s, xprev):
    def body(g_ref, id_ref, s_ref, xp_ref, o_ref):
        aggr = (g_ref[0] + g_ref[1]) * id_ref[...]
        o_ref[...] = jnp.maximum(aggr + s_ref[...], 0.0) + xp_ref[...]

    return pl.pallas_call(
        body,
        grid=(1,),
        in_specs=[
            pl.BlockSpec((NSC, N, HID), lambda i: (0, 0, 0)),
            pl.BlockSpec((N, 1), lambda i: (0, 0)),
            pl.BlockSpec((N, HID), lambda i: (0, 0)),
            pl.BlockSpec((N, HID), lambda i: (0, 0)),
        ],
        out_specs=pl.BlockSpec((N, HID), lambda i: (0, 0)),
        out_shape=jax.ShapeDtypeStruct((N, HID), jnp.float32),
    )(aggp, inv_deg2, s, xprev)


def _tc_batchmean(x):
    def body(x_ref, o_ref):
        o_ref[...] = jnp.mean(x_ref[...], axis=1, keepdims=True)

    npb = N // BATCH
    out = pl.pallas_call(
        body,
        grid=(BATCH,),
        in_specs=[pl.BlockSpec((1, npb, EMB), lambda b: (b, 0, 0))],
        out_specs=pl.BlockSpec((1, 1, EMB), lambda b: (b, 0, 0)),
        out_shape=jax.ShapeDtypeStruct((BATCH, 1, EMB), jnp.float32),
    )(x.reshape(BATCH, npb, EMB))
    return out.reshape(BATCH, EMB)


def kernel(x, edge_index, edge_type, batch_size, bn_gamma, bn_beta, relW, relb,
           normc, mpW, mpb, selfW, selfb):
    E = edge_index.shape[1]
    pad = EP - E

    xbn, xn = _tc_bn(x, bn_gamma, bn_beta)

    srcp = jnp.concatenate([edge_index[0], jnp.zeros((pad,), jnp.int32)])
    dstp = jnp.concatenate([edge_index[1], jnp.full((pad,), N, jnp.int32)])
    etp = jnp.concatenate([edge_type, jnp.zeros((pad,), jnp.int32)])
    gi_rel = etp * N + srcp
    frd_iv = etp * N + dstp
    frd_hist = jnp.concatenate(
        [frd_iv[:E], jnp.full((pad,), PAD_RD, jnp.int32)]
    )
    dst_deg = dstp + DEG_OFF

    pack_rel = jnp.stack(
        [gi_rel.reshape(TOTCH, CH), dstp.reshape(TOTCH, CH)], axis=1
    )
    pack_mp = jnp.stack(
        [srcp.reshape(TOTCH, CH), dstp.reshape(TOTCH, CH)], axis=1
    )
    pack_w = jnp.stack(
        [
            srcp.reshape(TOTCH, CH),
            dstp.reshape(TOTCH, CH),
            frd_iv.reshape(TOTCH, CH),
        ],
        axis=1,
    )

    hist = _sc_hist(frd_hist, dst_deg)
    inv_rel, inv_deg = _tc_invtab(hist, normc)
    alphas = _sc_wcos(xn, pack_w, inv_rel[0], inv_rel[1])
    inv_deg2 = inv_deg.reshape(N, 1)

    xcur = xbn
    for i in range(NL):
        ytab = _tc_ytab(xcur, relW[i], relb[i]).reshape(NREL * N, HID)
        accp = _sc_edge_rel(ytab, pack_rel, alphas[i])
        h = _tc_relucomb(accp)
        z, s = _tc_zs(h, mpW[i], mpb[i], selfW[i], selfb[i])
        aggp = _sc_edge_mp(z, pack_mp)
        xcur = _tc_comb2(aggp, inv_deg2, s, xcur)

    return _tc_batchmean(xcur)
